# baseline (speedup 1.0000x reference)
import functools
import jax
import jax.numpy as jnp
from jax import lax
from jax.experimental import pallas as pl
from jax.experimental.pallas import tpu as pltpu
from jax.experimental.pallas import tpu_sc as plsc

N_NODES = 10000
NUM_RBF = 32
CUT_HI = 5.0
TW = 256
BE = 800
NG = 10


GC = 128


def _sc_gather(t_src, t_dst, src, dst):
    ne = src.shape[0]
    nchunks = ne // GC
    kmax = (nchunks + 31) // 32

    @functools.partial(
        pl.kernel,
        mesh=plsc.VectorSubcoreMesh(core_axis_name="c", subcore_axis_name="s"),
        out_type=[jax.ShapeDtypeStruct((ne, TW), jnp.float32),
                  jax.ShapeDtypeStruct((ne, TW), jnp.float32)],
        scratch_types=[pltpu.VMEM((GC,), jnp.int32),
                       pltpu.VMEM((GC, TW), jnp.float32),
                       pltpu.VMEM((GC,), jnp.int32),
                       pltpu.VMEM((GC, TW), jnp.float32),
                       pltpu.SemaphoreType.DMA,
                       pltpu.SemaphoreType.DMA],
    )
    def gk(ts_hbm, td_hbm, src_hbm, dst_hbm, a_hbm, b_hbm,
           idx_a, rows_a, idx_b, rows_b, sem_a, sem_b):
        wid = lax.axis_index("s") * 2 + lax.axis_index("c")

        def body(k, _):
            j = wid + 32 * k

            @pl.when(j < nchunks)
            def _():
                e0 = j * GC
                pltpu.sync_copy(src_hbm.at[pl.ds(e0, GC)], idx_a)
                pltpu.sync_copy(dst_hbm.at[pl.ds(e0, GC)], idx_b)
                ca = pltpu.async_copy(ts_hbm.at[idx_a], rows_a, sem_a)
                cb = pltpu.async_copy(td_hbm.at[idx_b], rows_b, sem_b)
                ca.wait()
                cb.wait()
                pltpu.sync_copy(rows_a, a_hbm.at[pl.ds(e0, GC)])
                pltpu.sync_copy(rows_b, b_hbm.at[pl.ds(e0, GC)])
            return _

        lax.fori_loop(0, kmax, body, None, unroll=False)

    return gk(t_src, t_dst, src, dst)


def _sc_scatter(parts, zeros_nk, n_nodes):
    ns = len(parts)
    nch = [p.shape[0] // GC for p, _ in parts]
    rows_per_s = n_nodes // 16

    @functools.partial(
        pl.kernel,
        mesh=plsc.VectorSubcoreMesh(core_axis_name="c", subcore_axis_name="s"),
        out_type=jax.ShapeDtypeStruct((n_nodes, 512), jnp.float32),
        scratch_types=[pltpu.VMEM_SHARED((n_nodes, 128), jnp.float32),
                       pltpu.VMEM((GC,), jnp.int32),
                       pltpu.VMEM((GC, 128), jnp.float32)],
    )
    def sk(*refs):
        pays = [refs[2 * t] for t in range(ns)]
        srcs = [refs[2 * t + 1] for t in range(ns)]
        zero_hbm = refs[2 * ns]
        out_hbm = refs[2 * ns + 1]
        acc_sh, idx_v, rows_v = refs[2 * ns + 2:]
        c = lax.axis_index("c")
        s = lax.axis_index("s")
        r0 = s * rows_per_s
        for p_local in range(2):
            col0 = (c * 2 + p_local) * 128
            pltpu.sync_copy(zero_hbm.at[pl.ds(r0, rows_per_s)],
                            acc_sh.at[pl.ds(r0, rows_per_s)])
            plsc.subcore_barrier()

            for t in range(ns):
                pay_hbm, src_hbm, nchunks = pays[t], srcs[t], nch[t]

                def body(k, _, pay_hbm=pay_hbm, src_hbm=src_hbm,
                         nchunks=nchunks):
                    j = s + 16 * k

                    @pl.when(j < nchunks)
                    def _():
                        e0 = j * GC
                        pltpu.sync_copy(src_hbm.at[pl.ds(e0, GC)], idx_v)
                        pltpu.sync_copy(
                            pay_hbm.at[pl.ds(e0, GC), pl.ds(col0, 128)],
                            rows_v)
                        pltpu.sync_copy(rows_v, acc_sh.at[idx_v], add=True)
                    return _

                lax.fori_loop(0, (nchunks + 15) // 16, body, None,
                              unroll=False)
            plsc.subcore_barrier()
            pltpu.sync_copy(acc_sh.at[pl.ds(r0, rows_per_s)],
                            out_hbm.at[pl.ds(r0, rows_per_s),
                                       pl.ds(col0, 128)])
            plsc.subcore_barrier()

    flat = []
    for p, s_ in parts:
        flat += [p, s_]
    return sk(*flat, zeros_nk)


def _tc_edge_body(a_ref, b_ref, ev_ref, wdp_ref, m1t_ref, mrows_ref, t16_ref,
                  ebp_ref, wexp_ref, out_ref):
    uv = a_ref[...] + b_ref[...]
    ev = ev_ref[:, 0:3]
    w2 = jnp.sum(ev * ev, axis=1, keepdims=True)
    w = jnp.sqrt(w2)
    w3 = w * w * w
    ws = jnp.sqrt(w)
    c = 0.5 * (jnp.cos(w * (jnp.pi / CUT_HI)) + 1.0)
    c = c * (w < CUT_HI).astype(jnp.float32)

    step = CUT_HI / (NUM_RBF - 1)
    coeff = -0.5 / (step * step)
    off = lax.broadcasted_iota(jnp.int32, (1, NUM_RBF), 1).astype(jnp.float32) * step
    d1 = w - off
    d2 = w3 - off
    attr = jnp.concatenate([jnp.exp(coeff * d1 * d1),
                            jnp.exp(coeff * d2 * d2)], axis=1)

    h = attr @ wdp_ref[...] + mrows_ref[4:5, :]
    hc = h * c
    gamma = (uv + hc @ m1t_ref[...]
             + w3 * mrows_ref[0:1, :]
             + ws * mrows_ref[1:2, :]
             + w * mrows_ref[2:3, :]
             + mrows_ref[3:4, :])

    t16 = t16_ref[...]
    lane = lax.broadcasted_iota(jnp.int32, (1, 16), 1)
    gmask = lane < NG
    d = 1.0 / jnp.maximum(jnp.abs(w - t16), 1e-8)
    d = jnp.where(gmask, d, -1e30)
    e = jnp.exp(d - jnp.max(d, axis=1, keepdims=True))
    sm = e / jnp.sum(e, axis=1, keepdims=True)

    el = sm @ ebp_ref[...]
    for g in range(NG):
        el = el + sm[:, g:g + 1] * (gamma @ wexp_ref[g])

    evn = ev / w
    out_ref[:, 0:128] = el
    for d3 in range(3):
        out_ref[:, 128 * (d3 + 1):128 * (d3 + 2)] = evn[:, d3:d3 + 1] * el


def _tc_edge_pipeline(a_rows, b_rows, ev8, wdp_t, m1t, mrows, t16, ebp,
                      wexp_t):
    ne = a_rows.shape[0]
    grid = ne // BE
    return pl.pallas_call(
        _tc_edge_body,
        grid=(grid,),
        in_specs=[
            pl.BlockSpec((BE, TW), lambda i: (i, 0)),
            pl.BlockSpec((BE, TW), lambda i: (i, 0)),
            pl.BlockSpec((BE, 8), lambda i: (i, 0)),
            pl.BlockSpec((64, 256), lambda i: (0, 0)),
            pl.BlockSpec((256, 256), lambda i: (0, 0)),
            pl.BlockSpec((8, 256), lambda i: (0, 0)),
            pl.BlockSpec((1, 16), lambda i: (0, 0)),
            pl.BlockSpec((16, 128), lambda i: (0, 0)),
            pl.BlockSpec((NG, 256, 128), lambda i: (0, 0, 0)),
        ],
        out_specs=pl.BlockSpec((BE, 512), lambda i: (i, 0)),
        out_shape=jax.ShapeDtypeStruct((ne, 512), jnp.float32),
    )(a_rows, b_rows, ev8, wdp_t, m1t, mrows, t16, ebp, wexp_t)


def kernel(z, pos, batch, edge_index, emb, W_dp, b_dp, W_dij, b_dij,
           W_ai, b_ai, W_aj, b_aj, W_gam, b_gam, t_params,
           expert_W, expert_b):
    f32 = jnp.float32
    src = edge_index[0, ::2].astype(jnp.int32)
    dst = edge_index[1, ::2].astype(jnp.int32)
    ne = src.shape[0]

    Wg1 = W_gam[:, 0:128]
    Wg2 = W_gam[:, 128:384]
    Wg3 = W_gam[:, 384:512]
    U_emb = (emb @ W_ai.T + b_ai) @ Wg1.T
    V_emb = (emb @ W_aj.T + b_aj) @ Wg2.T
    Wd3 = Wg3 @ W_dij
    m1t = Wd3[:, 0:256].T
    bias_g = b_gam + Wg3 @ b_dij
    mrows = jnp.zeros((8, 256), f32)
    mrows = mrows.at[0].set(Wd3[:, 256])
    mrows = mrows.at[1].set(Wd3[:, 257])
    mrows = mrows.at[2].set(Wd3[:, 258])
    mrows = mrows.at[3].set(bias_g)
    mrows = mrows.at[4].set(b_dp)
    t16 = jnp.zeros((1, 16), f32).at[0, :NG].set(t_params)
    ebp = jnp.zeros((16, 128), f32).at[:NG].set(expert_b)
    wexp_t = expert_W.transpose(0, 2, 1)

    posf = pos.astype(f32)
    t_src = U_emb[z]
    t_dst = V_emb[z]
    pos8 = jnp.pad(posf, ((0, 0), (0, 5)))
    ev8_all = pos8[src] - pos8[dst]

    bounds = [0, 12800, 25600, 38400, 51200, 64000, 80000]
    parts = []
    for t in range(len(bounds) - 1):
        o, n = bounds[t], bounds[t + 1] - bounds[t]
        s_t = lax.dynamic_slice_in_dim(src, o, n)
        d_t = lax.dynamic_slice_in_dim(dst, o, n)
        a_t, b_t = _sc_gather(t_src, t_dst, s_t, d_t)
        ev_t = lax.dynamic_slice_in_dim(ev8_all, o, n)
        pay_t = _tc_edge_pipeline(a_t, b_t, ev_t, W_dp.T, m1t, mrows,
                                  t16, ebp, wexp_t)
        parts.append((pay_t, s_t))

    n_pad = 10240
    zeros_nk = jnp.zeros((n_pad, 128), f32)
    out_a = _sc_scatter(parts[:3], zeros_nk, n_pad)
    out_b = _sc_scatter(parts[3:], zeros_nk, n_pad)
    out = out_a[:z.shape[0]] + out_b[:z.shape[0]]

    atom_x = out[:, 0:128]
    vec = out[:, 128:512].reshape(z.shape[0], 3, 128)
    return (atom_x, vec, z, pos, batch)


# speedup vs baseline: 1.4743x; 1.4743x over previous
import functools
import jax
import jax.numpy as jnp
from jax import lax
from jax.experimental import pallas as pl
from jax.experimental.pallas import tpu as pltpu
from jax.experimental.pallas import tpu_sc as plsc

N_NODES = 10000
NUM_RBF = 32
CUT_HI = 5.0
TW = 384
BE = 800
NG = 10


GC = 128


def _sc_gather(t_src, t_dst, src, dst):
    ne = src.shape[0]
    nchunks = ne // GC
    kmax = (nchunks + 31) // 32

    @functools.partial(
        pl.kernel,
        mesh=plsc.VectorSubcoreMesh(core_axis_name="c", subcore_axis_name="s"),
        out_type=[jax.ShapeDtypeStruct((ne, TW), jnp.float32),
                  jax.ShapeDtypeStruct((ne, TW), jnp.float32)],
        scratch_types=[pltpu.VMEM((GC,), jnp.int32),
                       pltpu.VMEM((GC, TW), jnp.float32),
                       pltpu.VMEM((GC,), jnp.int32),
                       pltpu.VMEM((GC, TW), jnp.float32),
                       pltpu.SemaphoreType.DMA,
                       pltpu.SemaphoreType.DMA],
    )
    def gk(ts_hbm, td_hbm, src_hbm, dst_hbm, a_hbm, b_hbm,
           idx_a, rows_a, idx_b, rows_b, sem_a, sem_b):
        wid = lax.axis_index("s") * 2 + lax.axis_index("c")

        def body(k, _):
            j = wid + 32 * k

            @pl.when(j < nchunks)
            def _():
                e0 = j * GC
                pltpu.sync_copy(src_hbm.at[pl.ds(e0, GC)], idx_a)
                pltpu.sync_copy(dst_hbm.at[pl.ds(e0, GC)], idx_b)
                ca = pltpu.async_copy(ts_hbm.at[idx_a], rows_a, sem_a)
                cb = pltpu.async_copy(td_hbm.at[idx_b], rows_b, sem_b)
                ca.wait()
                cb.wait()
                pltpu.sync_copy(rows_a, a_hbm.at[pl.ds(e0, GC)])
                pltpu.sync_copy(rows_b, b_hbm.at[pl.ds(e0, GC)])
            return _

        lax.fori_loop(0, kmax, body, None, unroll=False)

    return gk(t_src, t_dst, src, dst)


def _sc_scatter(parts, zeros_nk, n_nodes):
    ns = len(parts)
    nch = [p.shape[0] // GC for p, _ in parts]
    rows_per_s = n_nodes // 16

    steps = []
    for t in range(ns):
        for k in range((nch[t] + 15) // 16):
            steps.append((t, k))

    @functools.partial(
        pl.kernel,
        mesh=plsc.VectorSubcoreMesh(core_axis_name="c", subcore_axis_name="s"),
        out_type=jax.ShapeDtypeStruct((n_nodes, 512), jnp.float32),
        scratch_types=[pltpu.VMEM_SHARED((n_nodes, 128), jnp.float32),
                       pltpu.VMEM((GC,), jnp.int32),
                       pltpu.VMEM((GC,), jnp.int32),
                       pltpu.VMEM((GC, 128), jnp.float32),
                       pltpu.VMEM((GC, 128), jnp.float32),
                       pltpu.SemaphoreType.DMA,
                       pltpu.SemaphoreType.DMA],
    )
    def sk(*refs):
        pays = [refs[2 * t] for t in range(ns)]
        srcs = [refs[2 * t + 1] for t in range(ns)]
        zero_hbm = refs[2 * ns]
        out_hbm = refs[2 * ns + 1]
        acc_sh = refs[2 * ns + 2]
        idx_v = refs[2 * ns + 3:2 * ns + 5]
        rows_v = refs[2 * ns + 5:2 * ns + 7]
        sems = refs[2 * ns + 7:2 * ns + 9]
        c = lax.axis_index("c")
        s = lax.axis_index("s")
        r0 = s * rows_per_s

        def start(i, col0):
            t, k = steps[i]
            p = i % 2
            j = s + 16 * k

            @pl.when(j < nch[t])
            def _():
                e0 = j * GC
                pltpu.sync_copy(srcs[t].at[pl.ds(e0, GC)], idx_v[p])
                pltpu.async_copy(
                    pays[t].at[pl.ds(e0, GC), pl.ds(col0, 128)],
                    rows_v[p], sems[p])

        def drain(i, col0):
            t, k = steps[i]
            p = i % 2
            j = s + 16 * k

            @pl.when(j < nch[t])
            def _():
                e0 = j * GC
                pltpu.make_async_copy(
                    pays[t].at[pl.ds(e0, GC), pl.ds(col0, 128)],
                    rows_v[p], sems[p]).wait()
                pltpu.sync_copy(rows_v[p], acc_sh.at[idx_v[p]], add=True)

        for p_local in range(2):
            col0 = (c * 2 + p_local) * 128
            pltpu.sync_copy(zero_hbm.at[pl.ds(r0, rows_per_s)],
                            acc_sh.at[pl.ds(r0, rows_per_s)])
            plsc.subcore_barrier()

            start(0, col0)
            for i in range(1, len(steps)):
                start(i, col0)
                drain(i - 1, col0)
            drain(len(steps) - 1, col0)

            plsc.subcore_barrier()
            pltpu.sync_copy(acc_sh.at[pl.ds(r0, rows_per_s)],
                            out_hbm.at[pl.ds(r0, rows_per_s),
                                       pl.ds(col0, 128)])
            plsc.subcore_barrier()

    flat = []
    for p, s_ in parts:
        flat += [p, s_]
    return sk(*flat, zeros_nk)


def _tc_edge_body(a_ref, b_ref, wdp_ref, m1t_ref, mrows_ref, t16_ref,
                  ebp_ref, wexp_ref, out_ref):
    s = a_ref[...] + b_ref[...]
    uv = s[:, 0:256]
    ev = s[:, 256:259]
    w2 = jnp.sum(ev * ev, axis=1, keepdims=True)
    w = jnp.sqrt(w2)
    w3 = w * w * w
    ws = jnp.sqrt(w)
    c = 0.5 * (jnp.cos(w * (jnp.pi / CUT_HI)) + 1.0)
    c = c * (w < CUT_HI).astype(jnp.float32)

    step = CUT_HI / (NUM_RBF - 1)
    coeff = -0.5 / (step * step)
    off = lax.broadcasted_iota(jnp.int32, (1, NUM_RBF), 1).astype(jnp.float32) * step
    d1 = w - off
    d2 = w3 - off
    attr = jnp.concatenate([jnp.exp(coeff * d1 * d1),
                            jnp.exp(coeff * d2 * d2)], axis=1)

    h = attr @ wdp_ref[...] + mrows_ref[4:5, :]
    hc = h * c
    gamma = (uv + hc @ m1t_ref[...]
             + w3 * mrows_ref[0:1, :]
             + ws * mrows_ref[1:2, :]
             + w * mrows_ref[2:3, :]
             + mrows_ref[3:4, :])

    t16 = t16_ref[...]
    lane = lax.broadcasted_iota(jnp.int32, (1, 16), 1)
    gmask = lane < NG
    d = 1.0 / jnp.maximum(jnp.abs(w - t16), 1e-8)
    d = jnp.where(gmask, d, -1e30)
    e = jnp.exp(d - jnp.max(d, axis=1, keepdims=True))
    sm = e / jnp.sum(e, axis=1, keepdims=True)

    el = sm @ ebp_ref[...]
    for g in range(NG):
        el = el + sm[:, g:g + 1] * (gamma @ wexp_ref[g])

    evn = ev / w
    out_ref[:, 0:128] = el
    for d3 in range(3):
        out_ref[:, 128 * (d3 + 1):128 * (d3 + 2)] = evn[:, d3:d3 + 1] * el


def _tc_edge_pipeline(a_rows, b_rows, wdp_t, m1t, mrows, t16, ebp, wexp_t):
    ne = a_rows.shape[0]
    grid = ne // BE
    return pl.pallas_call(
        _tc_edge_body,
        grid=(grid,),
        in_specs=[
            pl.BlockSpec((BE, TW), lambda i: (i, 0)),
            pl.BlockSpec((BE, TW), lambda i: (i, 0)),
            pl.BlockSpec((64, 256), lambda i: (0, 0)),
            pl.BlockSpec((256, 256), lambda i: (0, 0)),
            pl.BlockSpec((8, 256), lambda i: (0, 0)),
            pl.BlockSpec((1, 16), lambda i: (0, 0)),
            pl.BlockSpec((16, 128), lambda i: (0, 0)),
            pl.BlockSpec((NG, 256, 128), lambda i: (0, 0, 0)),
        ],
        out_specs=pl.BlockSpec((BE, 512), lambda i: (i, 0)),
        out_shape=jax.ShapeDtypeStruct((ne, 512), jnp.float32),
    )(a_rows, b_rows, wdp_t, m1t, mrows, t16, ebp, wexp_t)


def kernel(z, pos, batch, edge_index, emb, W_dp, b_dp, W_dij, b_dij,
           W_ai, b_ai, W_aj, b_aj, W_gam, b_gam, t_params,
           expert_W, expert_b):
    f32 = jnp.float32
    src = edge_index[0, ::2].astype(jnp.int32)
    dst = edge_index[1, ::2].astype(jnp.int32)
    ne = src.shape[0]

    Wg1 = W_gam[:, 0:128]
    Wg2 = W_gam[:, 128:384]
    Wg3 = W_gam[:, 384:512]
    U_emb = (emb @ W_ai.T + b_ai) @ Wg1.T
    V_emb = (emb @ W_aj.T + b_aj) @ Wg2.T
    Wd3 = Wg3 @ W_dij
    m1t = Wd3[:, 0:256].T
    bias_g = b_gam + Wg3 @ b_dij
    mrows = jnp.zeros((8, 256), f32)
    mrows = mrows.at[0].set(Wd3[:, 256])
    mrows = mrows.at[1].set(Wd3[:, 257])
    mrows = mrows.at[2].set(Wd3[:, 258])
    mrows = mrows.at[3].set(bias_g)
    mrows = mrows.at[4].set(b_dp)
    t16 = jnp.zeros((1, 16), f32).at[0, :NG].set(t_params)
    ebp = jnp.zeros((16, 128), f32).at[:NG].set(expert_b)
    wexp_t = expert_W.transpose(0, 2, 1)

    posf = pos.astype(f32)
    pad = jnp.zeros((z.shape[0], TW - 259), f32)
    t_src = jnp.concatenate([U_emb[z], posf, pad], axis=1)
    t_dst = jnp.concatenate([V_emb[z], -posf, pad], axis=1)

    bounds = [0, 12800, 25600, 38400, 51200, 64000, 80000]
    parts = []
    for t in range(len(bounds) - 1):
        o, n = bounds[t], bounds[t + 1] - bounds[t]
        s_t = lax.dynamic_slice_in_dim(src, o, n)
        d_t = lax.dynamic_slice_in_dim(dst, o, n)
        a_t, b_t = _sc_gather(t_src, t_dst, s_t, d_t)
        pay_t = _tc_edge_pipeline(a_t, b_t, W_dp.T, m1t, mrows,
                                  t16, ebp, wexp_t)
        parts.append((pay_t, s_t))

    n_pad = 10240
    zeros_nk = jnp.zeros((n_pad, 128), f32)
    out_a = _sc_scatter(parts[:3], zeros_nk, n_pad)
    out_b = _sc_scatter(parts[3:], zeros_nk, n_pad)
    out = out_a[:z.shape[0]] + out_b[:z.shape[0]]

    atom_x = out[:, 0:128]
    vec = out[:, 128:512].reshape(z.shape[0], 3, 128)
    return (atom_x, vec, z, pos, batch)
